# baseline (device time: 171418 ns/iter reference)
import jax
import jax.numpy as jnp
from jax import lax
from jax.experimental import pallas as pl
from jax.experimental.pallas import tpu as pltpu

C = 640


def kernel(x, assign, W1, W2):
    T, D = x.shape
    E, _, F = W1.shape

    my_y = lax.axis_index("y")

    owner = assign // 2
    to_peer = (owner != my_y).astype(jnp.int32)
    perm = jnp.argsort(to_peer)
    loc_idx = perm[:C]
    send_idx = perm[T - C :]

    xl = x[loc_idx].astype(jnp.bfloat16)
    as_l = assign[loc_idx].reshape(C, 1)
    xs = x[send_idx].astype(jnp.bfloat16)
    as_s = assign[send_idx].reshape(C, 1)

    def peer_of():
        return (
            lax.axis_index("x"),
            1 - lax.axis_index("y"),
            lax.axis_index("z"),
        )

    def peer_barrier(peer):
        barrier_sem = pltpu.get_barrier_semaphore()
        pl.semaphore_signal(
            barrier_sem,
            inc=1,
            device_id=peer,
            device_id_type=pl.DeviceIdType.MESH,
        )
        pl.semaphore_wait(barrier_sem, 1)

    def gather_body(xs_ref, as_ref, xr_ref, ar_ref, send_sems, recv_sems):
        peer = peer_of()
        peer_barrier(peer)

        rdma_x = pltpu.make_async_remote_copy(
            src_ref=xs_ref,
            dst_ref=xr_ref,
            send_sem=send_sems.at[0],
            recv_sem=recv_sems.at[0],
            device_id=peer,
            device_id_type=pl.DeviceIdType.MESH,
        )
        rdma_a = pltpu.make_async_remote_copy(
            src_ref=as_ref,
            dst_ref=ar_ref,
            send_sem=send_sems.at[1],
            recv_sem=recv_sems.at[1],
            device_id=peer,
            device_id_type=pl.DeviceIdType.MESH,
        )
        rdma_x.start()
        rdma_a.start()
        rdma_x.wait()
        rdma_a.wait()

    xr, ar = pl.pallas_call(
        gather_body,
        out_shape=(
            jax.ShapeDtypeStruct((C, D), jnp.bfloat16),
            jax.ShapeDtypeStruct((C, 1), jnp.int32),
        ),
        in_specs=[
            pl.BlockSpec(memory_space=pltpu.VMEM),
            pl.BlockSpec(memory_space=pltpu.VMEM),
        ],
        out_specs=(
            pl.BlockSpec(memory_space=pltpu.VMEM),
            pl.BlockSpec(memory_space=pltpu.VMEM),
        ),
        scratch_shapes=[
            pltpu.SemaphoreType.DMA((2,)),
            pltpu.SemaphoreType.DMA((2,)),
        ],
        compiler_params=pltpu.CompilerParams(collective_id=0),
    )(xs, as_s)

    FT = 512
    NFT = F // FT

    def moe_body(
        xl_ref, al_ref, xr_ref, ar_ref, w1_ref, w2_ref,
        out_ref, xm_ref, acc_ref,
    ):
        e = pl.program_id(0)
        ft = pl.program_id(1)

        @pl.when(ft == 0)
        def _():
            e_glob = lax.axis_index("y") * E + e
            zero = jnp.bfloat16(0)
            xm_ref[0:C] = jnp.where(al_ref[...] == e_glob, xl_ref[...], zero)
            xm_ref[C : 2 * C] = jnp.where(
                ar_ref[...] == e_glob, xr_ref[...], zero
            )

        w1 = w1_ref[0].astype(jnp.bfloat16)
        w2 = w2_ref[0].astype(jnp.bfloat16)
        h = jnp.dot(xm_ref[...], w1, preferred_element_type=jnp.float32)
        h = jnp.maximum(h, 0.0).astype(jnp.bfloat16)
        part = jnp.dot(h, w2, preferred_element_type=jnp.float32)

        @pl.when((e == 0) & (ft == 0))
        def _():
            acc_ref[...] = part

        @pl.when((e != 0) | (ft != 0))
        def _():
            acc_ref[...] = acc_ref[...] + part

        @pl.when((e == E - 1) & (ft == NFT - 1))
        def _():
            out_ref[0] = acc_ref[0:C].astype(jnp.bfloat16)
            out_ref[1] = acc_ref[C : 2 * C].astype(jnp.bfloat16)

    contrib = pl.pallas_call(
        moe_body,
        grid=(E, NFT),
        out_shape=jax.ShapeDtypeStruct((2, C, D), jnp.bfloat16),
        in_specs=[
            pl.BlockSpec((C, D), lambda e, ft: (0, 0)),
            pl.BlockSpec((C, 1), lambda e, ft: (0, 0)),
            pl.BlockSpec((C, D), lambda e, ft: (0, 0)),
            pl.BlockSpec((C, 1), lambda e, ft: (0, 0)),
            pl.BlockSpec((1, D, FT), lambda e, ft: (e, 0, ft)),
            pl.BlockSpec((1, FT, D), lambda e, ft: (e, ft, 0)),
        ],
        out_specs=pl.BlockSpec((2, C, D), lambda e, ft: (0, 0, 0)),
        scratch_shapes=[
            pltpu.VMEM((2 * C, D), jnp.bfloat16),
            pltpu.VMEM((2 * C, D), jnp.float32),
        ],
        compiler_params=pltpu.CompilerParams(
            dimension_semantics=("arbitrary", "arbitrary")
        ),
    )(xl, as_l, xr, ar, W1, W2)

    def reduce_body(c_ref, stacked_ref, send_sem, recv_sem):
        peer = peer_of()
        peer_barrier(peer)

        rdma = pltpu.make_async_remote_copy(
            src_ref=c_ref.at[1],
            dst_ref=stacked_ref.at[1],
            send_sem=send_sem,
            recv_sem=recv_sem,
            device_id=peer,
            device_id_type=pl.DeviceIdType.MESH,
        )
        rdma.start()
        stacked_ref[0] = c_ref[0]
        rdma.wait()

    stacked = pl.pallas_call(
        reduce_body,
        out_shape=jax.ShapeDtypeStruct((2, C, D), jnp.bfloat16),
        in_specs=[pl.BlockSpec(memory_space=pltpu.VMEM)],
        out_specs=pl.BlockSpec(memory_space=pltpu.VMEM),
        scratch_shapes=[
            pltpu.SemaphoreType.DMA,
            pltpu.SemaphoreType.DMA,
        ],
        compiler_params=pltpu.CompilerParams(collective_id=1),
    )(contrib)

    pos = jnp.arange(C, dtype=jnp.int32)
    valid_l = owner[loc_idx] == my_y
    valid_s = owner[send_idx] != my_y
    inv = (
        jnp.zeros((T,), jnp.int32)
        .at[loc_idx]
        .add(jnp.where(valid_l, pos + 1, 0))
        .at[send_idx]
        .add(jnp.where(valid_s, pos + 1 + C, 0))
    )
    out = stacked.reshape(2 * C, D)[inv - 1].astype(jnp.float32)
    return out


# device time: 100306 ns/iter; 1.7090x vs baseline; 1.7090x over previous
import jax
import jax.numpy as jnp
from jax import lax
from jax.experimental import pallas as pl
from jax.experimental.pallas import tpu as pltpu

C = 640


def kernel(x, assign, W1, W2):
    T, D = x.shape
    E, _, F = W1.shape

    my_y = lax.axis_index("y")

    owner = assign // 2
    to_peer = (owner != my_y).astype(jnp.int32)
    perm = jnp.argsort(to_peer)
    loc_idx = perm[:C]
    send_idx = perm[T - C :]

    xl = x[loc_idx].astype(jnp.bfloat16)
    as_l = assign[loc_idx].reshape(C, 1)
    xs = x[send_idx].astype(jnp.bfloat16)
    as_s = assign[send_idx].reshape(C, 1)

    def peer_of():
        return (
            lax.axis_index("x"),
            1 - lax.axis_index("y"),
            lax.axis_index("z"),
        )

    def peer_barrier(peer):
        barrier_sem = pltpu.get_barrier_semaphore()
        pl.semaphore_signal(
            barrier_sem,
            inc=1,
            device_id=peer,
            device_id_type=pl.DeviceIdType.MESH,
        )
        pl.semaphore_wait(barrier_sem, 1)

    def gather_body(xs_ref, as_ref, xr_ref, ar_ref, send_sems, recv_sems):
        peer = peer_of()
        peer_barrier(peer)

        rdma_x = pltpu.make_async_remote_copy(
            src_ref=xs_ref,
            dst_ref=xr_ref,
            send_sem=send_sems.at[0],
            recv_sem=recv_sems.at[0],
            device_id=peer,
            device_id_type=pl.DeviceIdType.MESH,
        )
        rdma_a = pltpu.make_async_remote_copy(
            src_ref=as_ref,
            dst_ref=ar_ref,
            send_sem=send_sems.at[1],
            recv_sem=recv_sems.at[1],
            device_id=peer,
            device_id_type=pl.DeviceIdType.MESH,
        )
        rdma_x.start()
        rdma_a.start()
        rdma_x.wait()
        rdma_a.wait()

    xr, ar = pl.pallas_call(
        gather_body,
        out_shape=(
            jax.ShapeDtypeStruct((C, D), jnp.bfloat16),
            jax.ShapeDtypeStruct((C, 1), jnp.int32),
        ),
        in_specs=[
            pl.BlockSpec(memory_space=pltpu.VMEM),
            pl.BlockSpec(memory_space=pltpu.VMEM),
        ],
        out_specs=(
            pl.BlockSpec(memory_space=pltpu.VMEM),
            pl.BlockSpec(memory_space=pltpu.VMEM),
        ),
        scratch_shapes=[
            pltpu.SemaphoreType.DMA((2,)),
            pltpu.SemaphoreType.DMA((2,)),
        ],
        compiler_params=pltpu.CompilerParams(collective_id=0),
    )(xs, as_s)

    FT = 512
    NFT = F // FT

    def moe_body(
        xl_ref, al_ref, xr_ref, ar_ref, w1_ref, w2_ref,
        out_ref, xm_ref, acc_ref,
    ):
        e = pl.program_id(0)
        ft = pl.program_id(1)

        @pl.when(ft == 0)
        def _():
            e_glob = lax.axis_index("y") * E + e
            zero = jnp.bfloat16(0)
            xm_ref[0:C] = jnp.where(al_ref[...] == e_glob, xl_ref[...], zero)
            xm_ref[C : 2 * C] = jnp.where(
                ar_ref[...] == e_glob, xr_ref[...], zero
            )

        w1 = w1_ref[0].astype(jnp.bfloat16)
        w2 = w2_ref[0].astype(jnp.bfloat16)
        h = jnp.dot(xm_ref[...], w1, preferred_element_type=jnp.float32)
        h = jnp.maximum(h, 0.0).astype(jnp.bfloat16)
        part = jnp.dot(h, w2, preferred_element_type=jnp.float32)

        @pl.when((e == 0) & (ft == 0))
        def _():
            acc_ref[...] = part

        @pl.when((e != 0) | (ft != 0))
        def _():
            acc_ref[...] = acc_ref[...] + part

        @pl.when((e == E - 1) & (ft == NFT - 1))
        def _():
            out_ref[0] = acc_ref[0:C].astype(jnp.bfloat16)
            out_ref[1] = acc_ref[C : 2 * C].astype(jnp.bfloat16)

    contrib = pl.pallas_call(
        moe_body,
        grid=(E, NFT),
        out_shape=jax.ShapeDtypeStruct((2, C, D), jnp.bfloat16),
        in_specs=[
            pl.BlockSpec((C, D), lambda e, ft: (0, 0)),
            pl.BlockSpec((C, 1), lambda e, ft: (0, 0)),
            pl.BlockSpec((C, D), lambda e, ft: (0, 0)),
            pl.BlockSpec((C, 1), lambda e, ft: (0, 0)),
            pl.BlockSpec((1, D, FT), lambda e, ft: (e, 0, ft)),
            pl.BlockSpec((1, FT, D), lambda e, ft: (e, ft, 0)),
        ],
        out_specs=pl.BlockSpec((2, C, D), lambda e, ft: (0, 0, 0)),
        scratch_shapes=[
            pltpu.VMEM((2 * C, D), jnp.bfloat16),
            pltpu.VMEM((2 * C, D), jnp.float32),
        ],
        compiler_params=pltpu.CompilerParams(
            dimension_semantics=("arbitrary", "arbitrary")
        ),
    )(xl, as_l, xr, ar, W1, W2)

    def reduce_body(c_ref, stacked_ref, send_sem, recv_sem):
        peer = peer_of()
        peer_barrier(peer)

        rdma = pltpu.make_async_remote_copy(
            src_ref=c_ref.at[1],
            dst_ref=stacked_ref.at[1],
            send_sem=send_sem,
            recv_sem=recv_sem,
            device_id=peer,
            device_id_type=pl.DeviceIdType.MESH,
        )
        rdma.start()
        stacked_ref[0] = c_ref[0]
        rdma.wait()

    stacked = pl.pallas_call(
        reduce_body,
        out_shape=jax.ShapeDtypeStruct((2, C, D), jnp.bfloat16),
        in_specs=[pl.BlockSpec(memory_space=pltpu.VMEM)],
        out_specs=pl.BlockSpec(memory_space=pltpu.VMEM),
        scratch_shapes=[
            pltpu.SemaphoreType.DMA,
            pltpu.SemaphoreType.DMA,
        ],
        compiler_params=pltpu.CompilerParams(collective_id=1),
    )(contrib)

    rank = jnp.argsort(perm).astype(jnp.int32)
    row = jnp.where(owner == my_y, rank, rank + 2 * C - T)
    out = stacked.reshape(2 * C, D)[row].astype(jnp.float32)
    return out
